# Initial kernel scaffold; baseline (speedup 1.0000x reference)
import jax
import jax.numpy as jnp
from jax.experimental import pallas as pl


def kernel(x, W):
    raise NotImplementedError("write your pallas kernel here")


# speedup vs baseline: 9.6690x; 9.6690x over previous
import functools

import numpy as np
import jax
import jax.numpy as jnp
from jax import lax
from jax.experimental import pallas as pl
from jax.experimental.pallas import tpu as pltpu
from jax.experimental.pallas import tpu_sc as plsc

F = 26
V = 104000
D = 16
B = 4096
NPAIR = (F * (F - 1)) // 2
ROW_IDX = 2 * NPAIR

NC, NS, L = 2, 16, 16
NW = NC * NS
ROWS_W = B // NW
G = 4
NG = ROWS_W // G
IDX_PER_G = G * ROW_IDX

_pi = np.array([i for i in range(F - 1) for j in range(i + 1, F)], np.int32)
_pj = np.array([j for i in range(F - 1) for j in range(i + 1, F)], np.int32)
_sel = np.concatenate([_pi, _pj])
_off = (np.concatenate([_pj, _pi]).astype(np.int64) * V).astype(np.int32)

_starts = []
for _base in (0, NPAIR):
    _s = 0
    while _s + L <= NPAIR:
        _starts.append(_base + _s)
        _s += L
    if _s < NPAIR:
        _starts.append(_base + NPAIR - L)
NCH = len(_starts)
SELC = np.stack([_sel[s:s + L] for s in _starts]).astype(np.int32)
OFFC = np.stack([_off[s:s + L] for s in _starts]).astype(np.int32)
DST = list(_starts)

_gchunks = [128] * (IDX_PER_G // 128)
if IDX_PER_G % 128:
    _gchunks.append(IDX_PER_G % 128)

_mesh = plsc.VectorSubcoreMesh(core_axis_name="c", subcore_axis_name="s",
                               num_cores=NC, num_subcores=NS)


@functools.partial(
    pl.kernel,
    out_type=jax.ShapeDtypeStruct((B * NPAIR * D,), jnp.float32),
    mesh=_mesh,
    scratch_types=[
        pltpu.VMEM((ROWS_W * F,), jnp.int32),
        pltpu.VMEM((NCH * L,), jnp.int32),
        pltpu.VMEM((NCH * L,), jnp.int32),
        pltpu.VMEM((IDX_PER_G,), jnp.int32),
        pltpu.VMEM((IDX_PER_G, D), jnp.float32),
        pltpu.VMEM((G * NPAIR * D,), jnp.float32),
        pltpu.SemaphoreType.DMA,
    ],
    compiler_params=pltpu.CompilerParams(needs_layout_passes=False,
                                         use_tc_tiling_on_sc=False),
)
def _ffm_kernel(xf, selc_h, offc_h, flatw, out,
                xw, selc, offc, idxb, rowsb, outb, gsem):
    wid = lax.axis_index("s") * NC + lax.axis_index("c")
    base_row = wid * ROWS_W
    pltpu.sync_copy(xf.at[pl.ds(base_row * F, ROWS_W * F)], xw)
    pltpu.sync_copy(selc_h, selc)
    pltpu.sync_copy(offc_h, offc)
    lane = lax.broadcasted_iota(jnp.int32, (L,), 0)

    def step(g, carry):
        for r in range(G):
            xoff = (g * G + r) * F
            robase = r * ROW_IDX
            for c in range(NCH):
                sv = selc[pl.ds(c * L, L)] + xoff
                xv = plsc.load_gather(xw, [sv])
                idxb[pl.ds(robase + DST[c], L)] = xv + offc[pl.ds(c * L, L)]
        cps = []
        pos = 0
        for n in _gchunks:
            cps.append(pltpu.async_copy(
                flatw.at[idxb.at[pl.ds(pos, n)]],
                rowsb.at[pl.ds(pos, n)], gsem))
            pos += n
        for cp in cps:
            cp.wait()
        for r in range(G):
            def mul(p, c, _r=r):
                arowv, off = c
                av = plsc.load_gather(rowsb, [arowv, lane])
                bv = plsc.load_gather(rowsb, [arowv + NPAIR, lane])
                outb[pl.ds(off, L)] = av * bv
                return (arowv + 1, off + L)

            lax.fori_loop(
                0, NPAIR, mul,
                (jnp.full((L,), r * ROW_IDX, jnp.int32),
                 jnp.int32(r * NPAIR * D)),
                unroll=4)
        pltpu.sync_copy(
            outb,
            out.at[pl.ds((base_row + g * G) * NPAIR * D, G * NPAIR * D)])
        return carry

    lax.fori_loop(0, NG, step, 0)


def kernel(x, W):
    xf = x.reshape(-1).astype(jnp.int32)
    flatw = W.reshape(F * V, D)
    out = _ffm_kernel(xf, jnp.asarray(SELC).reshape(-1),
                      jnp.asarray(OFFC).reshape(-1), flatw)
    return out.reshape(B, NPAIR, D)
